# Initial kernel scaffold; baseline (speedup 1.0000x reference)
import jax
import jax.numpy as jnp
from jax.experimental import pallas as pl


def kernel(z, embedding_weight):
    raise NotImplementedError("write your pallas kernel here")


# speedup vs baseline: 1.1371x; 1.1371x over previous
import functools

import jax
import jax.numpy as jnp
from jax import lax
from jax.experimental import pallas as pl
from jax.experimental.pallas import tpu as pltpu
from jax.experimental.pallas import tpu_sc as plsc

_B = 16384
_D = 32
_K = 8192
_BT = 128
_LOSS_SCALE = 12.5 / (_B * _D)


def _vq_tc_body(z_ref, e_ref, codes_ref, loss_ref, c_ref):
    pid = pl.program_id(0)
    nprog = pl.num_programs(0)

    @pl.when(pid == 0)
    def _init():
        e = e_ref[...]
        c_ref[0, :] = jnp.sum(e * e, axis=1)
        loss_ref[...] = jnp.zeros((1, 1), jnp.float32)

    z = z_ref[...]
    a = jnp.sum(z * z, axis=1, keepdims=True)
    m = lax.dot_general(
        z, e_ref[...], (((1,), (1,)), ((), ())),
        preferred_element_type=jnp.float32,
    )
    dist = (a - 2.0 * m) + c_ref[...]

    half = _K // 2
    d0 = dist[:, :half]
    d1 = dist[:, half:]
    m0 = jnp.min(d0, axis=1, keepdims=True)
    m1 = jnp.min(d1, axis=1, keepdims=True)
    ii = lax.broadcasted_iota(jnp.int32, d0.shape, 1)
    i0 = jnp.min(jnp.where(d0 == m0, ii, _K), axis=1)
    i1 = jnp.min(jnp.where(d1 == m1, ii, _K), axis=1) + half
    m0b = lax.convert_element_type(
        lax.convert_element_type(m0, jnp.bfloat16), jnp.float32)
    take1 = m1 < m0b
    codes_ref[...] = jnp.where(take1[:, 0], i1, i0)

    sel = jnp.where(take1, m1, m0)
    loss_ref[...] += jnp.sum(sel).reshape(1, 1)

    @pl.when(pid == nprog - 1)
    def _fin():
        loss_ref[...] = loss_ref[...] * _LOSS_SCALE


def _vq_tc(z, e):
    grid = (_B // _BT,)
    return pl.pallas_call(
        _vq_tc_body,
        grid=grid,
        in_specs=[
            pl.BlockSpec((_BT, _D), lambda i: (i, 0)),
            pl.BlockSpec((_K, _D), lambda i: (0, 0)),
        ],
        out_specs=[
            pl.BlockSpec((_BT,), lambda i: (i,)),
            pl.BlockSpec((1, 1), lambda i: (0, 0)),
        ],
        out_shape=[
            jax.ShapeDtypeStruct((_B,), jnp.int32),
            jax.ShapeDtypeStruct((1, 1), jnp.float32),
        ],
        scratch_shapes=[pltpu.VMEM((1, _K), jnp.float32)],
    )(z, e)


def _gather_sc(table, codes):
    info = plsc.get_sparse_core_info()
    nw = info.num_cores * info.num_subcores
    bpw = _B // nw
    mesh = plsc.VectorSubcoreMesh(core_axis_name="c", subcore_axis_name="s")

    @functools.partial(
        pl.kernel,
        out_type=jax.ShapeDtypeStruct((_B, _D), jnp.float32),
        mesh=mesh,
        scratch_types=[
            pltpu.VMEM((bpw // 128, 128), jnp.int32),
            pltpu.VMEM((bpw // 128, 128, _D), jnp.float32),
            pltpu.SemaphoreType.DMA,
        ],
        compiler_params=pltpu.CompilerParams(use_tc_tiling_on_sc=False),
    )
    def k(table_hbm, idx_hbm, out_hbm, idx_v, rows_v, sem):
        wid = lax.axis_index("s") * info.num_cores + lax.axis_index("c")
        base = wid * bpw
        nchunk = bpw // 128
        for j in range(nchunk):
            pltpu.sync_copy(idx_hbm.at[pl.ds(base + j * 128, 128)],
                            idx_v.at[j])
        copies = [pltpu.async_copy(table_hbm.at[idx_v.at[j]],
                                   rows_v.at[j], sem)
                  for j in range(nchunk)]
        for cp in copies:
            cp.wait()
        for j in range(nchunk):
            pltpu.sync_copy(rows_v.at[j],
                            out_hbm.at[pl.ds(base + j * 128, 128)])

    return k(table, codes)


def kernel(z, embedding_weight):
    codes, loss = _vq_tc(z, embedding_weight)
    z_q = _gather_sc(embedding_weight, codes)
    commit_loss = loss[0, 0]
    z_q_st = z + lax.stop_gradient(z_q - z)
    return (z_q_st, commit_loss, codes)


# speedup vs baseline: 1.2349x; 1.0859x over previous
import functools

import jax
import jax.numpy as jnp
from jax import lax
from jax.experimental import pallas as pl
from jax.experimental.pallas import tpu as pltpu
from jax.experimental.pallas import tpu_sc as plsc

_B = 16384
_D = 32
_K = 8192
_BT = 128
_LOSS_SCALE = 12.5 / (_B * _D)


def _vq_tc_body(z_ref, e_ref, codes_ref, loss_ref, c_ref, iota_ref):
    pid = pl.program_id(0)
    nprog = pl.num_programs(0)

    @pl.when(pid == 0)
    def _init():
        e = e_ref[...]
        c_ref[0, :] = jnp.sum(e * e, axis=1)
        iota_ref[...] = lax.convert_element_type(
            lax.broadcasted_iota(jnp.int32, (1, _K // 2), 1), jnp.float32)
        loss_ref[...] = jnp.zeros((1, 1), jnp.float32)

    z = z_ref[...]
    a = jnp.sum(z * z, axis=1, keepdims=True)
    m2 = lax.dot_general(
        z + z, e_ref[...], (((1,), (1,)), ((), ())),
        preferred_element_type=jnp.float32,
    )
    dist = (a - m2) + c_ref[...]

    half = _K // 2
    d0 = dist[:, :half]
    d1 = dist[:, half:]
    m0 = jnp.min(d0, axis=1, keepdims=True)
    m1 = jnp.min(d1, axis=1, keepdims=True)
    ii = iota_ref[...]
    i0f = jnp.min(jnp.where(d0 == m0, ii, float(_K)), axis=1)
    i1f = jnp.min(jnp.where(d1 == m1, ii, float(_K)), axis=1)
    i0 = lax.convert_element_type(i0f, jnp.int32)
    i1 = lax.convert_element_type(i1f, jnp.int32) + half
    m0b = lax.convert_element_type(
        lax.convert_element_type(m0, jnp.bfloat16), jnp.float32)
    take1 = m1 < m0b
    codes_ref[...] = jnp.where(take1[:, 0], i1, i0)

    sel = jnp.where(take1, m1, m0)
    loss_ref[...] += jnp.sum(sel).reshape(1, 1)

    @pl.when(pid == nprog - 1)
    def _fin():
        loss_ref[...] = loss_ref[...] * _LOSS_SCALE


def _vq_tc(z, e):
    grid = (_B // _BT,)
    return pl.pallas_call(
        _vq_tc_body,
        grid=grid,
        in_specs=[
            pl.BlockSpec((_BT, _D), lambda i: (i, 0)),
            pl.BlockSpec((_K, _D), lambda i: (0, 0)),
        ],
        out_specs=[
            pl.BlockSpec((_BT,), lambda i: (i,)),
            pl.BlockSpec((1, 1), lambda i: (0, 0)),
        ],
        out_shape=[
            jax.ShapeDtypeStruct((_B,), jnp.int32),
            jax.ShapeDtypeStruct((1, 1), jnp.float32),
        ],
        scratch_shapes=[pltpu.VMEM((1, _K), jnp.float32),
                        pltpu.VMEM((1, _K // 2), jnp.float32)],
    )(z, e)


def _gather_sc(table, codes):
    info = plsc.get_sparse_core_info()
    nw = info.num_cores * info.num_subcores
    bpw = _B // nw
    mesh = plsc.VectorSubcoreMesh(core_axis_name="c", subcore_axis_name="s")

    @functools.partial(
        pl.kernel,
        out_type=jax.ShapeDtypeStruct((_B, _D), jnp.float32),
        mesh=mesh,
        scratch_types=[
            pltpu.VMEM((bpw // 128, 128), jnp.int32),
            pltpu.VMEM((bpw // 128, 128, _D), jnp.float32),
            pltpu.SemaphoreType.DMA,
        ],
        compiler_params=pltpu.CompilerParams(use_tc_tiling_on_sc=False),
    )
    def k(table_hbm, idx_hbm, out_hbm, idx_v, rows_v, sem):
        wid = lax.axis_index("s") * info.num_cores + lax.axis_index("c")
        base = wid * bpw
        nchunk = bpw // 128
        for j in range(nchunk):
            pltpu.sync_copy(idx_hbm.at[pl.ds(base + j * 128, 128)],
                            idx_v.at[j])
        copies = [pltpu.async_copy(table_hbm.at[idx_v.at[j]],
                                   rows_v.at[j], sem)
                  for j in range(nchunk)]
        for cp in copies:
            cp.wait()
        for j in range(nchunk):
            pltpu.sync_copy(rows_v.at[j],
                            out_hbm.at[pl.ds(base + j * 128, 128)])

    return k(table, codes)


def kernel(z, embedding_weight):
    codes, loss = _vq_tc(z, embedding_weight)
    z_q = _gather_sc(embedding_weight, codes)
    commit_loss = loss[0, 0]
    z_q_st = z + lax.stop_gradient(z_q - z)
    return (z_q_st, commit_loss, codes)


# speedup vs baseline: 1.4281x; 1.1565x over previous
import functools

import jax
import jax.numpy as jnp
from jax import lax
from jax.experimental import pallas as pl
from jax.experimental.pallas import tpu as pltpu
from jax.experimental.pallas import tpu_sc as plsc

_B = 16384
_D = 32
_K = 8192
_BT = 256
_LOSS_SCALE = 12.5 / (_B * _D)


def _vq_tc_body(z_ref, e_ref, codes_ref, loss_ref, c_ref, iota_ref):
    pid = pl.program_id(0)
    nprog = pl.num_programs(0)

    @pl.when(pid == 0)
    def _init():
        e = e_ref[...]
        c_ref[0, :] = jnp.sum(e * e, axis=1)
        iota_ref[...] = lax.convert_element_type(
            lax.broadcasted_iota(jnp.int32, (1, _K // 2), 1), jnp.float32)
        loss_ref[...] = jnp.zeros((1, 1), jnp.float32)

    z = z_ref[...]
    a = jnp.sum(z * z, axis=1, keepdims=True)
    m2 = lax.dot_general(
        z + z, e_ref[...], (((1,), (1,)), ((), ())),
        preferred_element_type=jnp.float32,
    )
    dist = (a - m2) + c_ref[...]

    half = _K // 2
    d0 = dist[:, :half]
    d1 = dist[:, half:]
    m0 = jnp.min(d0, axis=1, keepdims=True)
    m1 = jnp.min(d1, axis=1, keepdims=True)
    ii = iota_ref[...]
    i0f = jnp.min(jnp.where(d0 == m0, ii, float(_K)), axis=1)
    i1f = jnp.min(jnp.where(d1 == m1, ii, float(_K)), axis=1)
    i0 = lax.convert_element_type(i0f, jnp.int32)
    i1 = lax.convert_element_type(i1f, jnp.int32) + half
    m0b = lax.convert_element_type(
        lax.convert_element_type(m0, jnp.bfloat16), jnp.float32)
    take1 = m1 < m0b
    codes_ref[...] = jnp.where(take1[:, 0], i1, i0)

    sel = jnp.where(take1, m1, m0)
    loss_ref[...] += jnp.sum(sel).reshape(1, 1)

    @pl.when(pid == nprog - 1)
    def _fin():
        loss_ref[...] = loss_ref[...] * _LOSS_SCALE


def _vq_tc(z, e):
    grid = (_B // _BT,)
    return pl.pallas_call(
        _vq_tc_body,
        grid=grid,
        in_specs=[
            pl.BlockSpec((_BT, _D), lambda i: (i, 0)),
            pl.BlockSpec((_K, _D), lambda i: (0, 0)),
        ],
        out_specs=[
            pl.BlockSpec((_BT,), lambda i: (i,)),
            pl.BlockSpec((1, 1), lambda i: (0, 0)),
        ],
        out_shape=[
            jax.ShapeDtypeStruct((_B,), jnp.int32),
            jax.ShapeDtypeStruct((1, 1), jnp.float32),
        ],
        scratch_shapes=[pltpu.VMEM((1, _K), jnp.float32),
                        pltpu.VMEM((1, _K // 2), jnp.float32)],
    )(z, e)


def _gather_sc(table, codes):
    info = plsc.get_sparse_core_info()
    nw = info.num_cores * info.num_subcores
    bpw = _B // nw
    mesh = plsc.VectorSubcoreMesh(core_axis_name="c", subcore_axis_name="s")

    @functools.partial(
        pl.kernel,
        out_type=jax.ShapeDtypeStruct((_B, _D), jnp.float32),
        mesh=mesh,
        scratch_types=[
            pltpu.VMEM((bpw // 128, 128), jnp.int32),
            pltpu.VMEM((bpw // 128, 128, _D), jnp.float32),
            pltpu.SemaphoreType.DMA,
        ],
        compiler_params=pltpu.CompilerParams(use_tc_tiling_on_sc=False),
    )
    def k(table_hbm, idx_hbm, out_hbm, idx_v, rows_v, sem):
        wid = lax.axis_index("s") * info.num_cores + lax.axis_index("c")
        base = wid * bpw
        nchunk = bpw // 128
        for j in range(nchunk):
            pltpu.sync_copy(idx_hbm.at[pl.ds(base + j * 128, 128)],
                            idx_v.at[j])
        copies = [pltpu.async_copy(table_hbm.at[idx_v.at[j]],
                                   rows_v.at[j], sem)
                  for j in range(nchunk)]
        for cp in copies:
            cp.wait()
        for j in range(nchunk):
            pltpu.sync_copy(rows_v.at[j],
                            out_hbm.at[pl.ds(base + j * 128, 128)])

    return k(table, codes)


def kernel(z, embedding_weight):
    codes, loss = _vq_tc(z, embedding_weight)
    z_q = _gather_sc(embedding_weight, codes)
    commit_loss = loss[0, 0]
    z_q_st = z + lax.stop_gradient(z_q - z)
    return (z_q_st, commit_loss, codes)


# speedup vs baseline: 1.4528x; 1.0173x over previous
import functools

import jax
import jax.numpy as jnp
from jax import lax
from jax.experimental import pallas as pl
from jax.experimental.pallas import tpu as pltpu
from jax.experimental.pallas import tpu_sc as plsc

_B = 16384
_D = 32
_K = 8192
_BT = 512
_LOSS_SCALE = 12.5 / (_B * _D)


def _vq_tc_body(z_ref, e_ref, codes_ref, loss_ref, c_ref, iota_ref):
    pid = pl.program_id(0)
    nprog = pl.num_programs(0)

    @pl.when(pid == 0)
    def _init():
        e = e_ref[...]
        c_ref[0, :] = jnp.sum(e * e, axis=1)
        iota_ref[...] = lax.convert_element_type(
            lax.broadcasted_iota(jnp.int32, (1, _K // 2), 1), jnp.float32)
        loss_ref[...] = jnp.zeros((1, 1), jnp.float32)

    z = z_ref[...]
    a = jnp.sum(z * z, axis=1, keepdims=True)
    m2 = lax.dot_general(
        z + z, e_ref[...], (((1,), (1,)), ((), ())),
        preferred_element_type=jnp.float32,
    )
    dist = (a - m2) + c_ref[...]

    half = _K // 2
    d0 = dist[:, :half]
    d1 = dist[:, half:]
    m0 = jnp.min(d0, axis=1, keepdims=True)
    m1 = jnp.min(d1, axis=1, keepdims=True)
    ii = iota_ref[...]
    i0f = jnp.min(jnp.where(d0 == m0, ii, float(_K)), axis=1)
    i1f = jnp.min(jnp.where(d1 == m1, ii, float(_K)), axis=1)
    i0 = lax.convert_element_type(i0f, jnp.int32)
    i1 = lax.convert_element_type(i1f, jnp.int32) + half
    m0b = lax.convert_element_type(
        lax.convert_element_type(m0, jnp.bfloat16), jnp.float32)
    take1 = m1 < m0b
    codes_ref[...] = jnp.where(take1[:, 0], i1, i0)

    sel = jnp.where(take1, m1, m0)
    loss_ref[...] += jnp.sum(sel).reshape(1, 1)

    @pl.when(pid == nprog - 1)
    def _fin():
        loss_ref[...] = loss_ref[...] * _LOSS_SCALE


def _vq_tc(z, e):
    grid = (_B // _BT,)
    return pl.pallas_call(
        _vq_tc_body,
        grid=grid,
        in_specs=[
            pl.BlockSpec((_BT, _D), lambda i: (i, 0)),
            pl.BlockSpec((_K, _D), lambda i: (0, 0)),
        ],
        out_specs=[
            pl.BlockSpec((_BT,), lambda i: (i,)),
            pl.BlockSpec((1, 1), lambda i: (0, 0)),
        ],
        out_shape=[
            jax.ShapeDtypeStruct((_B,), jnp.int32),
            jax.ShapeDtypeStruct((1, 1), jnp.float32),
        ],
        scratch_shapes=[pltpu.VMEM((1, _K), jnp.float32),
                        pltpu.VMEM((1, _K // 2), jnp.float32)],
    )(z, e)


def _gather_sc(table, codes):
    info = plsc.get_sparse_core_info()
    nw = info.num_cores * info.num_subcores
    bpw = _B // nw
    mesh = plsc.VectorSubcoreMesh(core_axis_name="c", subcore_axis_name="s")

    @functools.partial(
        pl.kernel,
        out_type=jax.ShapeDtypeStruct((_B, _D), jnp.float32),
        mesh=mesh,
        scratch_types=[
            pltpu.VMEM((bpw // 128, 128), jnp.int32),
            pltpu.VMEM((bpw // 128, 128, _D), jnp.float32),
            pltpu.SemaphoreType.DMA,
        ],
        compiler_params=pltpu.CompilerParams(use_tc_tiling_on_sc=False),
    )
    def k(table_hbm, idx_hbm, out_hbm, idx_v, rows_v, sem):
        wid = lax.axis_index("s") * info.num_cores + lax.axis_index("c")
        base = wid * bpw
        nchunk = bpw // 128
        for j in range(nchunk):
            pltpu.sync_copy(idx_hbm.at[pl.ds(base + j * 128, 128)],
                            idx_v.at[j])
        copies = [pltpu.async_copy(table_hbm.at[idx_v.at[j]],
                                   rows_v.at[j], sem)
                  for j in range(nchunk)]
        for cp in copies:
            cp.wait()
        for j in range(nchunk):
            pltpu.sync_copy(rows_v.at[j],
                            out_hbm.at[pl.ds(base + j * 128, 128)])

    return k(table, codes)


def kernel(z, embedding_weight):
    codes, loss = _vq_tc(z, embedding_weight)
    z_q = _gather_sc(embedding_weight, codes)
    commit_loss = loss[0, 0]
    z_q_st = z + lax.stop_gradient(z_q - z)
    return (z_q_st, commit_loss, codes)
